# Initial kernel scaffold; baseline (speedup 1.0000x reference)
import jax
import jax.numpy as jnp
from jax.experimental import pallas as pl


def kernel(x, edge_index, W_conv, b_conv, W0, Wt, W_ih, W_hh, b_ih, b_hh):
    raise NotImplementedError("write your pallas kernel here")


# speedup vs baseline: 10.7477x; 10.7477x over previous
import functools
import jax
import jax.numpy as jnp
from jax import lax
from jax.experimental import pallas as pl
from jax.experimental.pallas import tpu as pltpu
from jax.experimental.pallas import tpu_sc as plsc

N = 10000
E = 320000
D = 128
GH = 32
NC = 2
NS = 16
NW = NC * NS
EPW = E // NW
CH = 80
NCHUNK = EPW // CH
NP = 10240
SPS = NP // NS
ZR = 32

@functools.cache
def _mesh():
    return plsc.VectorSubcoreMesh(
        core_axis_name="c", subcore_axis_name="s",
        num_cores=NC, num_subcores=NS)


def _sc_counts_body(src_hbm, dst_hbm, out_hbm, src_buf, dst_buf, cs_loc, cd_loc):
    c = lax.axis_index("c")
    s = lax.axis_index("s")
    w = s * NC + c
    base = w * EPW
    pltpu.sync_copy(src_hbm.at[pl.ds(base, EPW)], src_buf)
    pltpu.sync_copy(dst_hbm.at[pl.ds(base, EPW)], dst_buf)
    zeros = jnp.zeros((16,), jnp.float32)

    def zero_body(i, _):
        cs_loc[pl.ds(i * 16, 16)] = zeros
        cd_loc[pl.ds(i * 16, 16)] = zeros
        return 0

    lax.fori_loop(0, N // 16, zero_body, 0)
    ones = jnp.ones((16,), jnp.float32)

    def body(i, _):
        si = src_buf[pl.ds(i * 16, 16)]
        plsc.addupdate_scatter(cs_loc, [si], ones)
        di = dst_buf[pl.ds(i * 16, 16)]
        plsc.addupdate_scatter(cd_loc, [di], ones)
        return 0

    lax.fori_loop(0, EPW // 16, body, 0)
    pltpu.sync_copy(cs_loc, out_hbm.at[0, w, 0])
    pltpu.sync_copy(cd_loc, out_hbm.at[1, w, 0])


@functools.cache
def _sc_counts():
    return pl.kernel(
        _sc_counts_body,
        out_type=jax.ShapeDtypeStruct((2, NW, 1, N), jnp.float32),
        mesh=_mesh(),
        compiler_params=pltpu.CompilerParams(needs_layout_passes=False),
        scratch_types=[
            pltpu.VMEM((EPW,), jnp.int32),
            pltpu.VMEM((EPW,), jnp.int32),
            pltpu.VMEM((N,), jnp.float32),
            pltpu.VMEM((N,), jnp.float32),
        ],
    )


def _sc_s_body(src_hbm, dst_hbm, dinv_hbm, s_out,
               dinv_loc, s_loc, src_buf, dst_buf):
    c = lax.axis_index("c")
    s = lax.axis_index("s")
    w = s * NC + c
    base = w * EPW
    pltpu.sync_copy(dinv_hbm, dinv_loc)
    pltpu.sync_copy(src_hbm.at[pl.ds(base, EPW)], src_buf)
    pltpu.sync_copy(dst_hbm.at[pl.ds(base, EPW)], dst_buf)
    zeros = jnp.zeros((16,), jnp.float32)

    def zs(i, _):
        s_loc[pl.ds(i * 16, 16)] = zeros
        return 0

    lax.fori_loop(0, N // 16, zs, 0)

    def body(i, _):
        idxs = src_buf[pl.ds(i * 16, 16)]
        vals = plsc.load_gather(dinv_loc, [idxs])
        idxd = dst_buf[pl.ds(i * 16, 16)]
        plsc.addupdate_scatter(s_loc, [idxd], vals)
        return 0

    lax.fori_loop(0, EPW // 16, body, 0)
    pltpu.sync_copy(s_loc, s_out.at[w, 0])


@functools.cache
def _sc_s():
    return pl.kernel(
        _sc_s_body,
        out_type=jax.ShapeDtypeStruct((NW, 1, N), jnp.float32),
        mesh=_mesh(),
        compiler_params=pltpu.CompilerParams(needs_layout_passes=False),
        scratch_types=[
            pltpu.VMEM((N,), jnp.float32),
            pltpu.VMEM((N,), jnp.float32),
            pltpu.VMEM((EPW,), jnp.int32),
            pltpu.VMEM((EPW,), jnp.int32),
        ],
    )


def _sc_feat_body(src_hbm, dst_hbm, y_hbm, acc_out, acc_sh,
                  src_buf, dstrow, yrow, zbuf):
    c = lax.axis_index("c")
    s = lax.axis_index("s")
    w = s * NC + c
    base = w * EPW
    pltpu.sync_copy(src_hbm.at[pl.ds(base, EPW)], src_buf)

    zeros = jnp.zeros((16,), jnp.float32)

    def zz(i, _):
        j = i // (D // 16)
        k = i % (D // 16)
        zbuf[j, pl.ds(k * 16, 16)] = zeros
        return 0

    lax.fori_loop(0, ZR * (D // 16), zz, 0)

    base_row = s * SPS
    for q in range(SPS // ZR):
        pltpu.sync_copy(zbuf, acc_sh.at[pl.ds(base_row + q * ZR, ZR)])
    plsc.subcore_barrier()

    def chunk(j, _):
        pltpu.sync_copy(dst_hbm.at[pl.ds(base + j * CH, CH)], dstrow.at[0])
        pltpu.sync_copy(y_hbm.at[src_buf.at[pl.ds(j * CH, CH)]], yrow)
        pltpu.sync_copy(yrow, acc_sh.at[dstrow.at[0]], add=True)
        return 0

    lax.fori_loop(0, NCHUNK, chunk, 0)
    plsc.subcore_barrier()
    pltpu.sync_copy(acc_sh.at[pl.ds(base_row, SPS)],
                    acc_out.at[c, pl.ds(base_row, SPS)])


@functools.cache
def _sc_feat():
    return pl.kernel(
        _sc_feat_body,
        out_type=jax.ShapeDtypeStruct((NC, NP, D), jnp.float32),
        mesh=_mesh(),
        compiler_params=pltpu.CompilerParams(needs_layout_passes=False),
        scratch_types=[
            pltpu.VMEM_SHARED((NP, D), jnp.float32),
            pltpu.VMEM((EPW,), jnp.int32),
            pltpu.VMEM((1, CH), jnp.int32),
            pltpu.VMEM((CH, D), jnp.float32),
            pltpu.VMEM((ZR, D), jnp.float32),
        ],
    )


def _tc_b1_body(counts_ref, dinv_ref, dinv2_ref, invcnt_ref):
    counts = counts_ref[...]
    cs = jnp.sum(counts[0], axis=0)
    cd = jnp.sum(counts[1], axis=0)
    deg_g = cs + cd
    deg = jnp.sqrt(deg_g + 1e-9)
    dinv = 1.0 / (deg + 1e-9)
    dinv2 = lax.rsqrt(jnp.maximum(cd + 1.0, 1e-12))
    invcnt = 1.0 / jnp.maximum(cd, 1.0)
    dinv_ref[...] = jnp.transpose(dinv)
    dinv2_ref[...] = jnp.transpose(dinv2)
    invcnt_ref[...] = jnp.transpose(invcnt)


def _tc_b1(counts):
    return pl.pallas_call(
        _tc_b1_body,
        out_shape=(
            jax.ShapeDtypeStruct((N, 1), jnp.float32),
            jax.ShapeDtypeStruct((N, 1), jnp.float32),
            jax.ShapeDtypeStruct((N, 1), jnp.float32),
        ),
    )(counts)


_RB = 1000


def _tc_b2_body(x_ref, wc_ref, w0_ref, dinv2_ref, y_ref, xs_ref):
    xb = x_ref[...]
    xw = jnp.dot(xb, wc_ref[...], preferred_element_type=jnp.float32)
    y_ref[...] = xw * dinv2_ref[...]
    xs_ref[...] = jnp.dot(xb, w0_ref[...], preferred_element_type=jnp.float32)


def _tc_b2(x, W_conv, W0, dinv2_c):
    grid = (N // _RB,)
    return pl.pallas_call(
        _tc_b2_body,
        grid=grid,
        in_specs=[
            pl.BlockSpec((_RB, D), lambda i: (i, 0)),
            pl.BlockSpec((D, D), lambda i: (0, 0)),
            pl.BlockSpec((D, D), lambda i: (0, 0)),
            pl.BlockSpec((_RB, 1), lambda i: (i, 0)),
        ],
        out_specs=[
            pl.BlockSpec((_RB, D), lambda i: (i, 0)),
            pl.BlockSpec((_RB, D), lambda i: (i, 0)),
        ],
        out_shape=(
            jax.ShapeDtypeStruct((N, D), jnp.float32),
            jax.ShapeDtypeStruct((N, D), jnp.float32),
        ),
    )(x, W_conv, W0, dinv2_c)


def _tc_sred_body(s_ref, dinv_ref, invcnt_ref, nn_ref):
    s_col = jnp.transpose(jnp.sum(s_ref[...], axis=0))
    nn_ref[...] = dinv_ref[...] * s_col * invcnt_ref[...]


def _tc_sred(s_parts, dinv_c, invcnt_c):
    return pl.pallas_call(
        _tc_sred_body,
        out_shape=jax.ShapeDtypeStruct((N, 1), jnp.float32),
    )(s_parts, dinv_c, invcnt_c)


def _tc_d1_body(acc0_ref, acc1_ref, y_ref, xs_ref, x_ref,
                nn_ref, dinv2_ref, wt_ref, wih_ref,
                bconv_ref, bih_ref, gi_ref):
    accs = acc0_ref[...] + acc1_ref[...]
    yb = y_ref[...]
    conv = dinv2_ref[...] * (accs + yb) + bconv_ref[...]
    h_neigh = jnp.maximum(conv * nn_ref[...], 0.0)
    xb = x_ref[...]
    hchg = jnp.dot(h_neigh - xb, wt_ref[...],
                   preferred_element_type=jnp.float32)
    hc = jax.nn.sigmoid(h_neigh + xs_ref[...] + hchg)
    gi_ref[...] = jnp.dot(hc, wih_ref[...],
                          preferred_element_type=jnp.float32) + bih_ref[...]


def _tc_d1(acc0, acc1, y, xs, x, nn_c, dinv2_c,
           Wt, WihT_pad, bconv2, bih_pad):
    grid = (N // _RB,)
    blk = pl.BlockSpec((_RB, D), lambda i: (i, 0))
    col = pl.BlockSpec((_RB, 1), lambda i: (i, 0))
    full = pl.BlockSpec((D, D), lambda i: (0, 0))
    return pl.pallas_call(
        _tc_d1_body,
        grid=grid,
        in_specs=[
            blk, blk, blk, blk, blk,
            col, col,
            full, full,
            pl.BlockSpec((1, D), lambda i: (0, 0)),
            pl.BlockSpec((1, D), lambda i: (0, 0)),
        ],
        out_specs=pl.BlockSpec((_RB, D), lambda i: (i, 0)),
        out_shape=jax.ShapeDtypeStruct((N, D), jnp.float32),
    )(acc0, acc1, y, xs, x, nn_c, dinv2_c,
      Wt, WihT_pad, bconv2, bih_pad)


def _tc_d2_body(gi_ref, whh_ref, bhh_ref, out_ref):
    whh = whh_ref[...]
    bhh = bhh_ref[...]

    def step(t, h):
        gi_t = gi_ref[pl.ds(t, 1), :]
        gh = jnp.dot(h, whh, preferred_element_type=jnp.float32) + bhh
        g = gi_t + gh
        sg = jax.nn.sigmoid(g)
        r64 = pltpu.roll(sg, 64, axis=1)
        ngf = jnp.tanh(gi_t + r64 * gh)
        zr = pltpu.roll(sg, 96, axis=1)
        ngr = pltpu.roll(ngf, 64, axis=1)
        h_new = (1.0 - zr) * ngr + zr * h
        out_ref[pl.ds(t, 1), :] = h_new[:, 0:GH]
        return h_new

    lax.fori_loop(0, N, step, jnp.zeros((1, D), jnp.float32))


def _tc_d2(gi, WhhT_pad, bhh_pad):
    return pl.pallas_call(
        _tc_d2_body,
        out_shape=jax.ShapeDtypeStruct((N, GH), jnp.float32),
    )(gi, WhhT_pad, bhh_pad)


def kernel(x, edge_index, W_conv, b_conv, W0, Wt, W_ih, W_hh, b_ih, b_hh):
    src, dst = edge_index[0], edge_index[1]
    counts = _sc_counts()(src, dst)
    dinv_c, dinv2_c, invcnt_c = _tc_b1(counts)
    y, xs = _tc_b2(x, W_conv, W0, dinv2_c)
    s_parts = _sc_s()(src, dst, dinv_c.reshape(N))
    acc_parts = _sc_feat()(src, dst, y)

    WihT_pad = jnp.zeros((D, D), jnp.float32).at[:, :3 * GH].set(W_ih.T)
    bih_pad = jnp.zeros((1, D), jnp.float32).at[0, :3 * GH].set(b_ih)
    WhhT_pad = jnp.zeros((D, D), jnp.float32).at[:GH, :3 * GH].set(W_hh.T)
    bhh_pad = jnp.zeros((1, D), jnp.float32).at[0, :3 * GH].set(b_hh)
    bconv2 = b_conv.reshape(1, D)

    nn_c = _tc_sred(s_parts, dinv_c, invcnt_c)
    gi = _tc_d1(acc_parts[0], acc_parts[1], y, xs, x, nn_c,
                dinv2_c, Wt, WihT_pad, bconv2, bih_pad)
    out = _tc_d2(gi, WhhT_pad, bhh_pad)
    return out


# speedup vs baseline: 21.0645x; 1.9599x over previous
import functools
import jax
import jax.numpy as jnp
from jax import lax
from jax.experimental import pallas as pl
from jax.experimental.pallas import tpu as pltpu
from jax.experimental.pallas import tpu_sc as plsc

N = 10000
E = 320000
D = 128
GH = 32
NC = 2
NS = 16
NW = NC * NS
EPW = E // NW
CH = 80
NCHUNK = EPW // CH
NP = 10240
SPS = NP // NS
ZR = 32

@functools.cache
def _mesh():
    return plsc.VectorSubcoreMesh(
        core_axis_name="c", subcore_axis_name="s",
        num_cores=NC, num_subcores=NS)


def _sc_counts_body(src_hbm, dst_hbm, out_hbm, src_buf, dst_buf, cs_loc, cd_loc):
    c = lax.axis_index("c")
    s = lax.axis_index("s")
    w = s * NC + c
    base = w * EPW
    pltpu.sync_copy(src_hbm.at[pl.ds(base, EPW)], src_buf)
    pltpu.sync_copy(dst_hbm.at[pl.ds(base, EPW)], dst_buf)
    zeros = jnp.zeros((16,), jnp.float32)

    def zero_body(i, _):
        cs_loc[pl.ds(i * 16, 16)] = zeros
        cd_loc[pl.ds(i * 16, 16)] = zeros
        return 0

    lax.fori_loop(0, N // 16, zero_body, 0)
    ones = jnp.ones((16,), jnp.float32)

    def body(i, _):
        si = src_buf[pl.ds(i * 16, 16)]
        plsc.addupdate_scatter(cs_loc, [si], ones)
        di = dst_buf[pl.ds(i * 16, 16)]
        plsc.addupdate_scatter(cd_loc, [di], ones)
        return 0

    lax.fori_loop(0, EPW // 16, body, 0)
    pltpu.sync_copy(cs_loc, out_hbm.at[0, w, 0])
    pltpu.sync_copy(cd_loc, out_hbm.at[1, w, 0])


@functools.cache
def _sc_counts():
    return pl.kernel(
        _sc_counts_body,
        out_type=jax.ShapeDtypeStruct((2, NW, 1, N), jnp.float32),
        mesh=_mesh(),
        compiler_params=pltpu.CompilerParams(needs_layout_passes=False),
        scratch_types=[
            pltpu.VMEM((EPW,), jnp.int32),
            pltpu.VMEM((EPW,), jnp.int32),
            pltpu.VMEM((N,), jnp.float32),
            pltpu.VMEM((N,), jnp.float32),
        ],
    )


def _sc_sf_body(src_hbm, dst_hbm, dinv_hbm, y_hbm, s_out, acc_out, acc_sh,
                dinv_loc, s_loc, srcrow, dstrow, yrow0, yrow1, zbuf,
                sem0, sem1):
    c = lax.axis_index("c")
    s = lax.axis_index("s")
    w = s * NC + c
    base = w * EPW
    pltpu.sync_copy(dinv_hbm, dinv_loc)

    zeros = jnp.zeros((16,), jnp.float32)

    def zs(i, _):
        s_loc[pl.ds(i * 16, 16)] = zeros
        return 0

    lax.fori_loop(0, N // 16, zs, 0)

    def zz(i, _):
        j = i // (D // 16)
        k = i % (D // 16)
        zbuf[j, pl.ds(k * 16, 16)] = zeros
        return 0

    lax.fori_loop(0, ZR * (D // 16), zz, 0)

    base_row = s * SPS
    for q in range(SPS // ZR):
        pltpu.sync_copy(zbuf, acc_sh.at[pl.ds(base_row + q * ZR, ZR)])
    plsc.subcore_barrier()

    def load_idx(cidx, slot):
        pltpu.sync_copy(src_hbm.at[pl.ds(base + cidx * CH, CH)],
                        srcrow.at[slot])
        pltpu.sync_copy(dst_hbm.at[pl.ds(base + cidx * CH, CH)],
                        dstrow.at[slot])

    def spass(slot):
        def sub(k, _):
            idxs = srcrow[slot, pl.ds(k * 16, 16)]
            vals = plsc.load_gather(dinv_loc, [idxs])
            idxd = dstrow[slot, pl.ds(k * 16, 16)]
            plsc.addupdate_scatter(s_loc, [idxd], vals)
            return 0

        lax.fori_loop(0, CH // 16, sub, 0)

    load_idx(0, 0)
    pltpu.async_copy(y_hbm.at[srcrow.at[0]], yrow0, sem0)

    def pair(m, _):
        c0 = 2 * m
        load_idx(c0 + 1, 1)
        pltpu.async_copy(y_hbm.at[srcrow.at[1]], yrow1, sem1)
        pltpu.make_async_copy(y_hbm.at[srcrow.at[0]], yrow0, sem0).wait()
        spass(0)
        pltpu.sync_copy(yrow0, acc_sh.at[dstrow.at[0]], add=True)
        load_idx(c0 + 2, 0)
        pltpu.async_copy(y_hbm.at[srcrow.at[0]], yrow0, sem0)
        pltpu.make_async_copy(y_hbm.at[srcrow.at[1]], yrow1, sem1).wait()
        spass(1)
        pltpu.sync_copy(yrow1, acc_sh.at[dstrow.at[1]], add=True)
        return 0

    lax.fori_loop(0, (NCHUNK - 1) // 2, pair, 0)
    pltpu.make_async_copy(y_hbm.at[srcrow.at[0]], yrow0, sem0).wait()
    spass(0)
    pltpu.sync_copy(yrow0, acc_sh.at[dstrow.at[0]], add=True)

    pltpu.sync_copy(s_loc, s_out.at[w, 0])
    plsc.subcore_barrier()
    pltpu.sync_copy(acc_sh.at[pl.ds(base_row, SPS)],
                    acc_out.at[c, pl.ds(base_row, SPS)])


@functools.cache
def _sc_sf():
    return pl.kernel(
        _sc_sf_body,
        out_type=(
            jax.ShapeDtypeStruct((NW, 1, N), jnp.float32),
            jax.ShapeDtypeStruct((NC, NP, D), jnp.float32),
        ),
        mesh=_mesh(),
        compiler_params=pltpu.CompilerParams(needs_layout_passes=False),
        scratch_types=[
            pltpu.VMEM_SHARED((NP, D), jnp.float32),
            pltpu.VMEM((N,), jnp.float32),
            pltpu.VMEM((N,), jnp.float32),
            pltpu.VMEM((2, CH), jnp.int32),
            pltpu.VMEM((2, CH), jnp.int32),
            pltpu.VMEM((CH, D), jnp.float32),
            pltpu.VMEM((CH, D), jnp.float32),
            pltpu.VMEM((ZR, D), jnp.float32),
            pltpu.SemaphoreType.DMA,
            pltpu.SemaphoreType.DMA,
        ],
    )


def _tc_b1_body(counts_ref, dinv_ref, dinv2_ref, invcnt_ref):
    counts = counts_ref[...]
    cs = jnp.sum(counts[0], axis=0)
    cd = jnp.sum(counts[1], axis=0)
    deg_g = cs + cd
    deg = jnp.sqrt(deg_g + 1e-9)
    dinv = 1.0 / (deg + 1e-9)
    dinv2 = lax.rsqrt(jnp.maximum(cd + 1.0, 1e-12))
    invcnt = 1.0 / jnp.maximum(cd, 1.0)
    dinv_ref[...] = jnp.transpose(dinv)
    dinv2_ref[...] = jnp.transpose(dinv2)
    invcnt_ref[...] = jnp.transpose(invcnt)


def _tc_b1(counts):
    return pl.pallas_call(
        _tc_b1_body,
        out_shape=(
            jax.ShapeDtypeStruct((N, 1), jnp.float32),
            jax.ShapeDtypeStruct((N, 1), jnp.float32),
            jax.ShapeDtypeStruct((N, 1), jnp.float32),
        ),
    )(counts)


_RB = 1000


def _tc_b2_body(x_ref, wc_ref, w0_ref, dinv2_ref, y_ref, xs_ref):
    xb = x_ref[...]
    xw = jnp.dot(xb, wc_ref[...], preferred_element_type=jnp.float32)
    y_ref[...] = xw * dinv2_ref[...]
    xs_ref[...] = jnp.dot(xb, w0_ref[...], preferred_element_type=jnp.float32)


def _tc_b2(x, W_conv, W0, dinv2_c):
    grid = (N // _RB,)
    return pl.pallas_call(
        _tc_b2_body,
        grid=grid,
        in_specs=[
            pl.BlockSpec((_RB, D), lambda i: (i, 0)),
            pl.BlockSpec((D, D), lambda i: (0, 0)),
            pl.BlockSpec((D, D), lambda i: (0, 0)),
            pl.BlockSpec((_RB, 1), lambda i: (i, 0)),
        ],
        out_specs=[
            pl.BlockSpec((_RB, D), lambda i: (i, 0)),
            pl.BlockSpec((_RB, D), lambda i: (i, 0)),
        ],
        out_shape=(
            jax.ShapeDtypeStruct((N, D), jnp.float32),
            jax.ShapeDtypeStruct((N, D), jnp.float32),
        ),
    )(x, W_conv, W0, dinv2_c)


def _tc_sred_body(s_ref, dinv_ref, invcnt_ref, nn_ref):
    s_col = jnp.transpose(jnp.sum(s_ref[...], axis=0))
    nn_ref[...] = dinv_ref[...] * s_col * invcnt_ref[...]


def _tc_sred(s_parts, dinv_c, invcnt_c):
    return pl.pallas_call(
        _tc_sred_body,
        out_shape=jax.ShapeDtypeStruct((N, 1), jnp.float32),
    )(s_parts, dinv_c, invcnt_c)


def _tc_d1_body(acc0_ref, acc1_ref, y_ref, xs_ref, x_ref,
                nn_ref, dinv2_ref, wt_ref, wih_ref,
                bconv_ref, bsum_ref, gir_ref, giz_ref, gin_ref):
    accs = acc0_ref[...] + acc1_ref[...]
    yb = y_ref[...]
    conv = dinv2_ref[...] * (accs + yb) + bconv_ref[...]
    h_neigh = jnp.maximum(conv * nn_ref[...], 0.0)
    xb = x_ref[...]
    hchg = jnp.dot(h_neigh - xb, wt_ref[...],
                   preferred_element_type=jnp.float32)
    hc = jax.nn.sigmoid(h_neigh + xs_ref[...] + hchg)
    gi = jnp.dot(hc, wih_ref[...],
                 preferred_element_type=jnp.float32) + bsum_ref[...]
    gir_ref[...] = gi[:, 0:GH]
    giz_ref[...] = gi[:, GH:2 * GH]
    gin_ref[...] = gi[:, 2 * GH:3 * GH]


def _tc_d1(acc0, acc1, y, xs, x, nn_c, dinv2_c, Wt, WihT, bconv2, bsum):
    grid = (N // _RB,)
    blk = pl.BlockSpec((_RB, D), lambda i: (i, 0))
    col = pl.BlockSpec((_RB, 1), lambda i: (i, 0))
    gout = pl.BlockSpec((_RB, GH), lambda i: (i, 0))
    return pl.pallas_call(
        _tc_d1_body,
        grid=grid,
        in_specs=[
            blk, blk, blk, blk, blk,
            col, col,
            pl.BlockSpec((D, D), lambda i: (0, 0)),
            pl.BlockSpec((D, 3 * GH), lambda i: (0, 0)),
            pl.BlockSpec((1, D), lambda i: (0, 0)),
            pl.BlockSpec((1, 3 * GH), lambda i: (0, 0)),
        ],
        out_specs=[gout, gout, gout],
        out_shape=(
            jax.ShapeDtypeStruct((N, GH), jnp.float32),
            jax.ShapeDtypeStruct((N, GH), jnp.float32),
            jax.ShapeDtypeStruct((N, GH), jnp.float32),
        ),
    )(acc0, acc1, y, xs, x, nn_c, dinv2_c, Wt, WihT, bconv2, bsum)


def _tc_d2_body(gir_ref, giz_ref, gin_ref, wr_ref, wz_ref, wn_ref, bhn_ref,
                out_ref):
    wr = wr_ref[...]
    wz = wz_ref[...]
    wn = wn_ref[...]
    bhn = bhn_ref[...]

    def step(t, h):
        ghr = jnp.dot(h, wr, preferred_element_type=jnp.float32)
        ghz = jnp.dot(h, wz, preferred_element_type=jnp.float32)
        ghn = jnp.dot(h, wn, preferred_element_type=jnp.float32) + bhn
        q = 0.5 * ghn
        pre = gin_ref[pl.ds(t, 1), :] + q
        tr = jnp.tanh(0.5 * (gir_ref[pl.ds(t, 1), :] + ghr))
        tz = jnp.tanh(0.5 * (giz_ref[pl.ds(t, 1), :] + ghz))
        ng = jnp.tanh(pre + q * tr)
        z = 0.5 + 0.5 * tz
        h_new = ng + z * (h - ng)
        out_ref[pl.ds(t, 1), :] = h_new
        return h_new

    lax.fori_loop(0, N, step, jnp.zeros((1, GH), jnp.float32))


def _tc_d2(gir, giz, gin, Wr, Wz, Wn, bhn):
    return pl.pallas_call(
        _tc_d2_body,
        out_shape=jax.ShapeDtypeStruct((N, GH), jnp.float32),
    )(gir, giz, gin, Wr, Wz, Wn, bhn)


def kernel(x, edge_index, W_conv, b_conv, W0, Wt, W_ih, W_hh, b_ih, b_hh):
    src, dst = edge_index[0], edge_index[1]
    counts = _sc_counts()(src, dst)
    dinv_c, dinv2_c, invcnt_c = _tc_b1(counts)
    y, xs = _tc_b2(x, W_conv, W0, dinv2_c)
    s_parts, acc_parts = _sc_sf()(src, dst, dinv_c.reshape(N), y)

    WihT = W_ih.T
    bsum = jnp.concatenate(
        [b_ih[:2 * GH] + b_hh[:2 * GH], b_ih[2 * GH:]]).reshape(1, 3 * GH)
    bhn = b_hh[2 * GH:].reshape(1, GH)
    WhhT = W_hh.T
    Wr = WhhT[:, 0:GH]
    Wz = WhhT[:, GH:2 * GH]
    Wn = WhhT[:, 2 * GH:3 * GH]
    bconv2 = b_conv.reshape(1, D)

    nn_c = _tc_sred(s_parts, dinv_c, invcnt_c)
    gir, giz, gin = _tc_d1(acc_parts[0], acc_parts[1], y, xs, x, nn_c,
                           dinv2_c, Wt, WihT, bconv2, bsum)
    out = _tc_d2(gir, giz, gin, Wr, Wz, Wn, bhn)
    return out


# speedup vs baseline: 21.1934x; 1.0061x over previous
import functools
import jax
import jax.numpy as jnp
from jax import lax
from jax.experimental import pallas as pl
from jax.experimental.pallas import tpu as pltpu
from jax.experimental.pallas import tpu_sc as plsc

N = 10000
E = 320000
D = 128
GH = 32
NC = 2
NS = 16
NW = NC * NS
EPW = E // NW
CH = 80
NCHUNK = EPW // CH
NP = 10240
SPS = NP // NS
ZR = 32

@functools.cache
def _mesh():
    return plsc.VectorSubcoreMesh(
        core_axis_name="c", subcore_axis_name="s",
        num_cores=NC, num_subcores=NS)


def _sc_counts_body(src_hbm, dst_hbm, out_hbm, src_buf, dst_buf, cs_loc, cd_loc):
    c = lax.axis_index("c")
    s = lax.axis_index("s")
    w = s * NC + c
    base = w * EPW
    pltpu.sync_copy(src_hbm.at[pl.ds(base, EPW)], src_buf)
    pltpu.sync_copy(dst_hbm.at[pl.ds(base, EPW)], dst_buf)
    zeros = jnp.zeros((16,), jnp.float32)

    def zero_body(i, _):
        cs_loc[pl.ds(i * 16, 16)] = zeros
        cd_loc[pl.ds(i * 16, 16)] = zeros
        return 0

    lax.fori_loop(0, N // 16, zero_body, 0)
    ones = jnp.ones((16,), jnp.float32)

    def body(i, _):
        si = src_buf[pl.ds(i * 16, 16)]
        plsc.addupdate_scatter(cs_loc, [si], ones)
        di = dst_buf[pl.ds(i * 16, 16)]
        plsc.addupdate_scatter(cd_loc, [di], ones)
        return 0

    lax.fori_loop(0, EPW // 16, body, 0)
    pltpu.sync_copy(cs_loc, out_hbm.at[0, w, 0])
    pltpu.sync_copy(cd_loc, out_hbm.at[1, w, 0])


@functools.cache
def _sc_counts():
    return pl.kernel(
        _sc_counts_body,
        out_type=jax.ShapeDtypeStruct((2, NW, 1, N), jnp.float32),
        mesh=_mesh(),
        compiler_params=pltpu.CompilerParams(needs_layout_passes=False),
        scratch_types=[
            pltpu.VMEM((EPW,), jnp.int32),
            pltpu.VMEM((EPW,), jnp.int32),
            pltpu.VMEM((N,), jnp.float32),
            pltpu.VMEM((N,), jnp.float32),
        ],
    )


def _sc_sf_body(src_hbm, dst_hbm, dinv_hbm, y_hbm, s_out, acc_out, acc_sh,
                dinv_loc, s_loc, srcrow, dstrow, yrow0, yrow1, zbuf,
                sem0, sem1):
    c = lax.axis_index("c")
    s = lax.axis_index("s")
    w = s * NC + c
    base = w * EPW
    pltpu.sync_copy(dinv_hbm, dinv_loc)

    zeros = jnp.zeros((16,), jnp.float32)

    def zs(i, _):
        s_loc[pl.ds(i * 16, 16)] = zeros
        return 0

    lax.fori_loop(0, N // 16, zs, 0)

    def zz(i, _):
        j = i // (D // 16)
        k = i % (D // 16)
        zbuf[j, pl.ds(k * 16, 16)] = zeros
        return 0

    lax.fori_loop(0, ZR * (D // 16), zz, 0)

    base_row = s * SPS
    for q in range(SPS // ZR):
        pltpu.sync_copy(zbuf, acc_sh.at[pl.ds(base_row + q * ZR, ZR)])
    plsc.subcore_barrier()

    def load_idx(cidx, slot):
        pltpu.sync_copy(src_hbm.at[pl.ds(base + cidx * CH, CH)],
                        srcrow.at[slot])
        pltpu.sync_copy(dst_hbm.at[pl.ds(base + cidx * CH, CH)],
                        dstrow.at[slot])

    def spass(slot):
        def sub(k, _):
            idxs = srcrow[slot, pl.ds(k * 16, 16)]
            vals = plsc.load_gather(dinv_loc, [idxs])
            idxd = dstrow[slot, pl.ds(k * 16, 16)]
            plsc.addupdate_scatter(s_loc, [idxd], vals)
            return 0

        lax.fori_loop(0, CH // 16, sub, 0)

    load_idx(0, 0)
    pltpu.async_copy(y_hbm.at[srcrow.at[0]], yrow0, sem0)

    def pair(m, _):
        c0 = 2 * m
        load_idx(c0 + 1, 1)
        pltpu.async_copy(y_hbm.at[srcrow.at[1]], yrow1, sem1)
        pltpu.make_async_copy(y_hbm.at[srcrow.at[0]], yrow0, sem0).wait()
        spass(0)
        pltpu.sync_copy(yrow0, acc_sh.at[dstrow.at[0]], add=True)
        load_idx(c0 + 2, 0)
        pltpu.async_copy(y_hbm.at[srcrow.at[0]], yrow0, sem0)
        pltpu.make_async_copy(y_hbm.at[srcrow.at[1]], yrow1, sem1).wait()
        spass(1)
        pltpu.sync_copy(yrow1, acc_sh.at[dstrow.at[1]], add=True)
        return 0

    lax.fori_loop(0, (NCHUNK - 1) // 2, pair, 0)
    pltpu.make_async_copy(y_hbm.at[srcrow.at[0]], yrow0, sem0).wait()
    spass(0)
    pltpu.sync_copy(yrow0, acc_sh.at[dstrow.at[0]], add=True)

    pltpu.sync_copy(s_loc, s_out.at[w, 0])
    plsc.subcore_barrier()
    pltpu.sync_copy(acc_sh.at[pl.ds(base_row, SPS)],
                    acc_out.at[c, pl.ds(base_row, SPS)])


@functools.cache
def _sc_sf():
    return pl.kernel(
        _sc_sf_body,
        out_type=(
            jax.ShapeDtypeStruct((NW, 1, N), jnp.float32),
            jax.ShapeDtypeStruct((NC, NP, D), jnp.float32),
        ),
        mesh=_mesh(),
        compiler_params=pltpu.CompilerParams(needs_layout_passes=False),
        scratch_types=[
            pltpu.VMEM_SHARED((NP, D), jnp.float32),
            pltpu.VMEM((N,), jnp.float32),
            pltpu.VMEM((N,), jnp.float32),
            pltpu.VMEM((2, CH), jnp.int32),
            pltpu.VMEM((2, CH), jnp.int32),
            pltpu.VMEM((CH, D), jnp.float32),
            pltpu.VMEM((CH, D), jnp.float32),
            pltpu.VMEM((ZR, D), jnp.float32),
            pltpu.SemaphoreType.DMA,
            pltpu.SemaphoreType.DMA,
        ],
    )


def _tc_b1_body(counts_ref, dinv_ref, dinv2_ref, invcnt_ref):
    counts = counts_ref[...]
    cs = jnp.sum(counts[0], axis=0)
    cd = jnp.sum(counts[1], axis=0)
    deg_g = cs + cd
    deg = jnp.sqrt(deg_g + 1e-9)
    dinv = 1.0 / (deg + 1e-9)
    dinv2 = lax.rsqrt(jnp.maximum(cd + 1.0, 1e-12))
    invcnt = 1.0 / jnp.maximum(cd, 1.0)
    dinv_ref[...] = jnp.transpose(dinv)
    dinv2_ref[...] = jnp.transpose(dinv2)
    invcnt_ref[...] = jnp.transpose(invcnt)


def _tc_b1(counts):
    return pl.pallas_call(
        _tc_b1_body,
        out_shape=(
            jax.ShapeDtypeStruct((N, 1), jnp.float32),
            jax.ShapeDtypeStruct((N, 1), jnp.float32),
            jax.ShapeDtypeStruct((N, 1), jnp.float32),
        ),
    )(counts)


_RB = 1000


def _tc_b2_body(x_ref, wc_ref, w0_ref, dinv2_ref, y_ref, xs_ref):
    xb = x_ref[...]
    xw = jnp.dot(xb, wc_ref[...], preferred_element_type=jnp.float32)
    y_ref[...] = xw * dinv2_ref[...]
    xs_ref[...] = jnp.dot(xb, w0_ref[...], preferred_element_type=jnp.float32)


def _tc_b2(x, W_conv, W0, dinv2_c):
    grid = (N // _RB,)
    return pl.pallas_call(
        _tc_b2_body,
        grid=grid,
        in_specs=[
            pl.BlockSpec((_RB, D), lambda i: (i, 0)),
            pl.BlockSpec((D, D), lambda i: (0, 0)),
            pl.BlockSpec((D, D), lambda i: (0, 0)),
            pl.BlockSpec((_RB, 1), lambda i: (i, 0)),
        ],
        out_specs=[
            pl.BlockSpec((_RB, D), lambda i: (i, 0)),
            pl.BlockSpec((_RB, D), lambda i: (i, 0)),
        ],
        out_shape=(
            jax.ShapeDtypeStruct((N, D), jnp.float32),
            jax.ShapeDtypeStruct((N, D), jnp.float32),
        ),
    )(x, W_conv, W0, dinv2_c)


def _tc_sred_body(s_ref, dinv_ref, invcnt_ref, nn_ref):
    s_col = jnp.transpose(jnp.sum(s_ref[...], axis=0))
    nn_ref[...] = dinv_ref[...] * s_col * invcnt_ref[...]


def _tc_sred(s_parts, dinv_c, invcnt_c):
    return pl.pallas_call(
        _tc_sred_body,
        out_shape=jax.ShapeDtypeStruct((N, 1), jnp.float32),
    )(s_parts, dinv_c, invcnt_c)


def _tc_d1_body(acc0_ref, acc1_ref, y_ref, xs_ref, x_ref,
                nn_ref, dinv2_ref, wt_ref, wih_ref,
                bconv_ref, bsum_ref, gir_ref, giz_ref, gin_ref):
    accs = acc0_ref[...] + acc1_ref[...]
    yb = y_ref[...]
    conv = dinv2_ref[...] * (accs + yb) + bconv_ref[...]
    h_neigh = jnp.maximum(conv * nn_ref[...], 0.0)
    xb = x_ref[...]
    hchg = jnp.dot(h_neigh - xb, wt_ref[...],
                   preferred_element_type=jnp.float32)
    hc = jax.nn.sigmoid(h_neigh + xs_ref[...] + hchg)
    gi = jnp.dot(hc, wih_ref[...],
                 preferred_element_type=jnp.float32) + bsum_ref[...]
    gir_ref[...] = 0.5 * gi[:, 0:GH]
    giz_ref[...] = 0.5 * gi[:, GH:2 * GH]
    gin_ref[...] = gi[:, 2 * GH:3 * GH]


def _tc_d1(acc0, acc1, y, xs, x, nn_c, dinv2_c, Wt, WihT, bconv2, bsum):
    grid = (N // _RB,)
    blk = pl.BlockSpec((_RB, D), lambda i: (i, 0))
    col = pl.BlockSpec((_RB, 1), lambda i: (i, 0))
    gout = pl.BlockSpec((_RB, GH), lambda i: (i, 0))
    return pl.pallas_call(
        _tc_d1_body,
        grid=grid,
        in_specs=[
            blk, blk, blk, blk, blk,
            col, col,
            pl.BlockSpec((D, D), lambda i: (0, 0)),
            pl.BlockSpec((D, 3 * GH), lambda i: (0, 0)),
            pl.BlockSpec((1, D), lambda i: (0, 0)),
            pl.BlockSpec((1, 3 * GH), lambda i: (0, 0)),
        ],
        out_specs=[gout, gout, gout],
        out_shape=(
            jax.ShapeDtypeStruct((N, GH), jnp.float32),
            jax.ShapeDtypeStruct((N, GH), jnp.float32),
            jax.ShapeDtypeStruct((N, GH), jnp.float32),
        ),
    )(acc0, acc1, y, xs, x, nn_c, dinv2_c, Wt, WihT, bconv2, bsum)


def _tc_d2_body(gir_ref, giz_ref, gin_ref, wr_ref, wz_ref, wn_ref, bhn_ref,
                out_ref):
    wr = wr_ref[...]
    wz = wz_ref[...]
    wn = wn_ref[...]
    bhn = bhn_ref[...]

    def step(t, h):
        ghr = jnp.dot(h, wr, preferred_element_type=jnp.float32)
        ghz = jnp.dot(h, wz, preferred_element_type=jnp.float32)
        q = jnp.dot(h, wn, preferred_element_type=jnp.float32) + bhn
        pre = gin_ref[pl.ds(t, 1), :] + q
        tr = jnp.tanh(gir_ref[pl.ds(t, 1), :] + ghr)
        tz = jnp.tanh(giz_ref[pl.ds(t, 1), :] + ghz)
        ng = jnp.tanh(pre + q * tr)
        z = 0.5 + 0.5 * tz
        h_new = ng + z * (h - ng)
        out_ref[pl.ds(t, 1), :] = h_new
        return h_new

    lax.fori_loop(0, N, step, jnp.zeros((1, GH), jnp.float32))


def _tc_d2(gir, giz, gin, Wr, Wz, Wn, bhn):
    return pl.pallas_call(
        _tc_d2_body,
        out_shape=jax.ShapeDtypeStruct((N, GH), jnp.float32),
    )(gir, giz, gin, Wr, Wz, Wn, bhn)


def kernel(x, edge_index, W_conv, b_conv, W0, Wt, W_ih, W_hh, b_ih, b_hh):
    src, dst = edge_index[0], edge_index[1]
    counts = _sc_counts()(src, dst)
    dinv_c, dinv2_c, invcnt_c = _tc_b1(counts)
    y, xs = _tc_b2(x, W_conv, W0, dinv2_c)
    s_parts, acc_parts = _sc_sf()(src, dst, dinv_c.reshape(N), y)

    WihT = W_ih.T
    bsum = jnp.concatenate(
        [b_ih[:2 * GH] + b_hh[:2 * GH], b_ih[2 * GH:]]).reshape(1, 3 * GH)
    bhn = 0.5 * b_hh[2 * GH:].reshape(1, GH)
    WhhT = W_hh.T
    Wr = 0.5 * WhhT[:, 0:GH]
    Wz = 0.5 * WhhT[:, GH:2 * GH]
    Wn = 0.5 * WhhT[:, 2 * GH:3 * GH]
    bconv2 = b_conv.reshape(1, D)

    nn_c = _tc_sred(s_parts, dinv_c, invcnt_c)
    gir, giz, gin = _tc_d1(acc_parts[0], acc_parts[1], y, xs, x, nn_c,
                           dinv2_c, Wt, WihT, bconv2, bsum)
    out = _tc_d2(gir, giz, gin, Wr, Wz, Wn, bhn)
    return out


# speedup vs baseline: 21.5345x; 1.0161x over previous
import functools
import jax
import jax.numpy as jnp
from jax import lax
from jax.experimental import pallas as pl
from jax.experimental.pallas import tpu as pltpu
from jax.experimental.pallas import tpu_sc as plsc

N = 10000
E = 320000
D = 128
GH = 32
NC = 2
NS = 16
NW = NC * NS
EPW = E // NW
CH = 80
NCHUNK = EPW // CH
NP = 10240
SPS = NP // NS
ZR = 32

@functools.cache
def _mesh():
    return plsc.VectorSubcoreMesh(
        core_axis_name="c", subcore_axis_name="s",
        num_cores=NC, num_subcores=NS)


def _sc_counts_body(src_hbm, dst_hbm, out_hbm, src_buf, dst_buf, cs_loc, cd_loc):
    c = lax.axis_index("c")
    s = lax.axis_index("s")
    w = s * NC + c
    base = w * EPW
    pltpu.sync_copy(src_hbm.at[pl.ds(base, EPW)], src_buf)
    pltpu.sync_copy(dst_hbm.at[pl.ds(base, EPW)], dst_buf)
    zeros = jnp.zeros((16,), jnp.float32)

    def zero_body(i, _):
        cs_loc[pl.ds(i * 16, 16)] = zeros
        cd_loc[pl.ds(i * 16, 16)] = zeros
        return 0

    lax.fori_loop(0, N // 16, zero_body, 0)
    ones = jnp.ones((16,), jnp.float32)

    def body(i, _):
        si = src_buf[pl.ds(i * 16, 16)]
        plsc.addupdate_scatter(cs_loc, [si], ones)
        di = dst_buf[pl.ds(i * 16, 16)]
        plsc.addupdate_scatter(cd_loc, [di], ones)
        return 0

    lax.fori_loop(0, EPW // 16, body, 0)
    pltpu.sync_copy(cs_loc, out_hbm.at[0, w, 0])
    pltpu.sync_copy(cd_loc, out_hbm.at[1, w, 0])


@functools.cache
def _sc_counts():
    return pl.kernel(
        _sc_counts_body,
        out_type=jax.ShapeDtypeStruct((2, NW, 1, N), jnp.float32),
        mesh=_mesh(),
        compiler_params=pltpu.CompilerParams(needs_layout_passes=False),
        scratch_types=[
            pltpu.VMEM((EPW,), jnp.int32),
            pltpu.VMEM((EPW,), jnp.int32),
            pltpu.VMEM((N,), jnp.float32),
            pltpu.VMEM((N,), jnp.float32),
        ],
    )


def _sc_sf_body(src_hbm, dst_hbm, dinv_hbm, y_hbm, s_out, acc_out, acc_sh,
                dinv_loc, s_loc, srcrow, dstrow, yrow0, yrow1, zbuf,
                sem0, sem1):
    c = lax.axis_index("c")
    s = lax.axis_index("s")
    w = s * NC + c
    base = w * EPW
    pltpu.sync_copy(dinv_hbm, dinv_loc)

    zeros = jnp.zeros((16,), jnp.float32)

    def zs(i, _):
        s_loc[pl.ds(i * 16, 16)] = zeros
        return 0

    lax.fori_loop(0, N // 16, zs, 0)

    def zz(i, _):
        j = i // (D // 16)
        k = i % (D // 16)
        zbuf[j, pl.ds(k * 16, 16)] = zeros
        return 0

    lax.fori_loop(0, ZR * (D // 16), zz, 0)

    base_row = s * SPS
    for q in range(SPS // ZR):
        pltpu.sync_copy(zbuf, acc_sh.at[pl.ds(base_row + q * ZR, ZR)])
    plsc.subcore_barrier()

    def load_idx(cidx, slot):
        pltpu.sync_copy(src_hbm.at[pl.ds(base + cidx * CH, CH)],
                        srcrow.at[slot])
        pltpu.sync_copy(dst_hbm.at[pl.ds(base + cidx * CH, CH)],
                        dstrow.at[slot])

    def spass(slot):
        def sub(k, _):
            idxs = srcrow[slot, pl.ds(k * 16, 16)]
            vals = plsc.load_gather(dinv_loc, [idxs])
            idxd = dstrow[slot, pl.ds(k * 16, 16)]
            plsc.addupdate_scatter(s_loc, [idxd], vals)
            return 0

        lax.fori_loop(0, CH // 16, sub, 0)

    load_idx(0, 0)
    pltpu.async_copy(y_hbm.at[srcrow.at[0]], yrow0, sem0)

    def pair(m, _):
        c0 = 2 * m
        load_idx(c0 + 1, 1)
        pltpu.async_copy(y_hbm.at[srcrow.at[1]], yrow1, sem1)
        pltpu.make_async_copy(y_hbm.at[srcrow.at[0]], yrow0, sem0).wait()
        spass(0)
        pltpu.sync_copy(yrow0, acc_sh.at[dstrow.at[0]], add=True)
        load_idx(c0 + 2, 0)
        pltpu.async_copy(y_hbm.at[srcrow.at[0]], yrow0, sem0)
        pltpu.make_async_copy(y_hbm.at[srcrow.at[1]], yrow1, sem1).wait()
        spass(1)
        pltpu.sync_copy(yrow1, acc_sh.at[dstrow.at[1]], add=True)
        return 0

    lax.fori_loop(0, (NCHUNK - 1) // 2, pair, 0)
    pltpu.make_async_copy(y_hbm.at[srcrow.at[0]], yrow0, sem0).wait()
    spass(0)
    pltpu.sync_copy(yrow0, acc_sh.at[dstrow.at[0]], add=True)

    pltpu.sync_copy(s_loc, s_out.at[w, 0])
    plsc.subcore_barrier()
    pltpu.sync_copy(acc_sh.at[pl.ds(base_row, SPS)],
                    acc_out.at[c, pl.ds(base_row, SPS)])


@functools.cache
def _sc_sf():
    return pl.kernel(
        _sc_sf_body,
        out_type=(
            jax.ShapeDtypeStruct((NW, 1, N), jnp.float32),
            jax.ShapeDtypeStruct((NC, NP, D), jnp.float32),
        ),
        mesh=_mesh(),
        compiler_params=pltpu.CompilerParams(needs_layout_passes=False),
        scratch_types=[
            pltpu.VMEM_SHARED((NP, D), jnp.float32),
            pltpu.VMEM((N,), jnp.float32),
            pltpu.VMEM((N,), jnp.float32),
            pltpu.VMEM((2, CH), jnp.int32),
            pltpu.VMEM((2, CH), jnp.int32),
            pltpu.VMEM((CH, D), jnp.float32),
            pltpu.VMEM((CH, D), jnp.float32),
            pltpu.VMEM((ZR, D), jnp.float32),
            pltpu.SemaphoreType.DMA,
            pltpu.SemaphoreType.DMA,
        ],
    )


def _tc_b1_body(counts_ref, dinv_ref, dinv2_ref, invcnt_ref):
    counts = counts_ref[...]
    cs = jnp.sum(counts[0], axis=0)
    cd = jnp.sum(counts[1], axis=0)
    deg_g = cs + cd
    deg = jnp.sqrt(deg_g + 1e-9)
    dinv = 1.0 / (deg + 1e-9)
    dinv2 = lax.rsqrt(jnp.maximum(cd + 1.0, 1e-12))
    invcnt = 1.0 / jnp.maximum(cd, 1.0)
    dinv_ref[...] = jnp.transpose(dinv)
    dinv2_ref[...] = jnp.transpose(dinv2)
    invcnt_ref[...] = jnp.transpose(invcnt)


def _tc_b1(counts):
    return pl.pallas_call(
        _tc_b1_body,
        out_shape=(
            jax.ShapeDtypeStruct((N, 1), jnp.float32),
            jax.ShapeDtypeStruct((N, 1), jnp.float32),
            jax.ShapeDtypeStruct((N, 1), jnp.float32),
        ),
    )(counts)


_RB = 1000


def _tc_b2_body(x_ref, wc_ref, w0_ref, dinv2_ref, y_ref, xs_ref):
    xb = x_ref[...]
    xw = jnp.dot(xb, wc_ref[...], preferred_element_type=jnp.float32)
    y_ref[...] = xw * dinv2_ref[...]
    xs_ref[...] = jnp.dot(xb, w0_ref[...], preferred_element_type=jnp.float32)


def _tc_b2(x, W_conv, W0, dinv2_c):
    grid = (N // _RB,)
    return pl.pallas_call(
        _tc_b2_body,
        grid=grid,
        in_specs=[
            pl.BlockSpec((_RB, D), lambda i: (i, 0)),
            pl.BlockSpec((D, D), lambda i: (0, 0)),
            pl.BlockSpec((D, D), lambda i: (0, 0)),
            pl.BlockSpec((_RB, 1), lambda i: (i, 0)),
        ],
        out_specs=[
            pl.BlockSpec((_RB, D), lambda i: (i, 0)),
            pl.BlockSpec((_RB, D), lambda i: (i, 0)),
        ],
        out_shape=(
            jax.ShapeDtypeStruct((N, D), jnp.float32),
            jax.ShapeDtypeStruct((N, D), jnp.float32),
        ),
    )(x, W_conv, W0, dinv2_c)


def _tc_sred_body(s_ref, dinv_ref, invcnt_ref, nn_ref):
    s_col = jnp.transpose(jnp.sum(s_ref[...], axis=0))
    nn_ref[...] = dinv_ref[...] * s_col * invcnt_ref[...]


def _tc_sred(s_parts, dinv_c, invcnt_c):
    return pl.pallas_call(
        _tc_sred_body,
        out_shape=jax.ShapeDtypeStruct((N, 1), jnp.float32),
    )(s_parts, dinv_c, invcnt_c)


def _tc_d1_body(acc0_ref, acc1_ref, y_ref, xs_ref, x_ref,
                nn_ref, dinv2_ref, wt_ref, wih_ref,
                bconv_ref, bsum_ref, gir_ref, giz_ref, gin_ref):
    accs = acc0_ref[...] + acc1_ref[...]
    yb = y_ref[...]
    conv = dinv2_ref[...] * (accs + yb) + bconv_ref[...]
    h_neigh = jnp.maximum(conv * nn_ref[...], 0.0)
    xb = x_ref[...]
    hchg = jnp.dot(h_neigh - xb, wt_ref[...],
                   preferred_element_type=jnp.float32)
    hc = jax.nn.sigmoid(h_neigh + xs_ref[...] + hchg)
    gi = jnp.dot(hc, wih_ref[...],
                 preferred_element_type=jnp.float32) + bsum_ref[...]
    gir_ref[...] = 0.5 * gi[:, 0:GH]
    giz_ref[...] = 0.5 * gi[:, GH:2 * GH]
    gin_ref[...] = gi[:, 2 * GH:3 * GH]


def _tc_d1(acc0, acc1, y, xs, x, nn_c, dinv2_c, Wt, WihT, bconv2, bsum):
    grid = (N // _RB,)
    blk = pl.BlockSpec((_RB, D), lambda i: (i, 0))
    col = pl.BlockSpec((_RB, 1), lambda i: (i, 0))
    gout = pl.BlockSpec((_RB, GH), lambda i: (i, 0))
    return pl.pallas_call(
        _tc_d1_body,
        grid=grid,
        in_specs=[
            blk, blk, blk, blk, blk,
            col, col,
            pl.BlockSpec((D, D), lambda i: (0, 0)),
            pl.BlockSpec((D, 3 * GH), lambda i: (0, 0)),
            pl.BlockSpec((1, D), lambda i: (0, 0)),
            pl.BlockSpec((1, 3 * GH), lambda i: (0, 0)),
        ],
        out_specs=[gout, gout, gout],
        out_shape=(
            jax.ShapeDtypeStruct((N, GH), jnp.float32),
            jax.ShapeDtypeStruct((N, GH), jnp.float32),
            jax.ShapeDtypeStruct((N, GH), jnp.float32),
        ),
    )(acc0, acc1, y, xs, x, nn_c, dinv2_c, Wt, WihT, bconv2, bsum)


def _tc_d2_body(gir_ref, giz_ref, gin_ref, wr_ref, wz_ref, wn_ref, bhn_ref,
                out_ref):
    wr = wr_ref[...]
    wz = wz_ref[...]
    wn = wn_ref[...]
    bhn = bhn_ref[...]

    def step(t, h):
        ghr = jnp.dot(h, wr, preferred_element_type=jnp.float32)
        ghz = jnp.dot(h, wz, preferred_element_type=jnp.float32)
        q = jnp.dot(h, wn, preferred_element_type=jnp.float32) + bhn
        pre = gin_ref[pl.ds(t, 1), :] + q
        tr = jnp.tanh(gir_ref[pl.ds(t, 1), :] + ghr)
        tz = jnp.tanh(giz_ref[pl.ds(t, 1), :] + ghz)
        ng = jnp.tanh(pre + q * tr)
        z = 0.5 + 0.5 * tz
        h_new = ng + z * (h - ng)
        out_ref[pl.ds(t, 1), :] = h_new
        return h_new

    lax.fori_loop(0, N, step, jnp.zeros((1, GH), jnp.float32), unroll=2)


def _tc_d2(gir, giz, gin, Wr, Wz, Wn, bhn):
    return pl.pallas_call(
        _tc_d2_body,
        out_shape=jax.ShapeDtypeStruct((N, GH), jnp.float32),
    )(gir, giz, gin, Wr, Wz, Wn, bhn)


def kernel(x, edge_index, W_conv, b_conv, W0, Wt, W_ih, W_hh, b_ih, b_hh):
    src, dst = edge_index[0], edge_index[1]
    counts = _sc_counts()(src, dst)
    dinv_c, dinv2_c, invcnt_c = _tc_b1(counts)
    y, xs = _tc_b2(x, W_conv, W0, dinv2_c)
    s_parts, acc_parts = _sc_sf()(src, dst, dinv_c.reshape(N), y)

    WihT = W_ih.T
    bsum = jnp.concatenate(
        [b_ih[:2 * GH] + b_hh[:2 * GH], b_ih[2 * GH:]]).reshape(1, 3 * GH)
    bhn = 0.5 * b_hh[2 * GH:].reshape(1, GH)
    WhhT = W_hh.T
    Wr = 0.5 * WhhT[:, 0:GH]
    Wz = 0.5 * WhhT[:, GH:2 * GH]
    Wn = 0.5 * WhhT[:, 2 * GH:3 * GH]
    bconv2 = b_conv.reshape(1, D)

    nn_c = _tc_sred(s_parts, dinv_c, invcnt_c)
    gir, giz, gin = _tc_d1(acc_parts[0], acc_parts[1], y, xs, x, nn_c,
                           dinv2_c, Wt, WihT, bconv2, bsum)
    out = _tc_d2(gir, giz, gin, Wr, Wz, Wn, bhn)
    return out


# speedup vs baseline: 21.7097x; 1.0081x over previous
import functools
import jax
import jax.numpy as jnp
from jax import lax
from jax.experimental import pallas as pl
from jax.experimental.pallas import tpu as pltpu
from jax.experimental.pallas import tpu_sc as plsc

N = 10000
E = 320000
D = 128
GH = 32
NC = 2
NS = 16
NW = NC * NS
EPW = E // NW
CH = 80
NCHUNK = EPW // CH
NP = 10240
SPS = NP // NS
ZR = 32

@functools.cache
def _mesh():
    return plsc.VectorSubcoreMesh(
        core_axis_name="c", subcore_axis_name="s",
        num_cores=NC, num_subcores=NS)


def _sc_counts_body(src_hbm, dst_hbm, out_hbm, src_buf, dst_buf, cs_loc, cd_loc):
    c = lax.axis_index("c")
    s = lax.axis_index("s")
    w = s * NC + c
    base = w * EPW
    pltpu.sync_copy(src_hbm.at[pl.ds(base, EPW)], src_buf)
    pltpu.sync_copy(dst_hbm.at[pl.ds(base, EPW)], dst_buf)
    zeros = jnp.zeros((16,), jnp.float32)

    def zero_body(i, _):
        cs_loc[pl.ds(i * 16, 16)] = zeros
        cd_loc[pl.ds(i * 16, 16)] = zeros
        return 0

    lax.fori_loop(0, N // 16, zero_body, 0)
    ones = jnp.ones((16,), jnp.float32)

    def body(i, _):
        si = src_buf[pl.ds(i * 16, 16)]
        plsc.addupdate_scatter(cs_loc, [si], ones)
        di = dst_buf[pl.ds(i * 16, 16)]
        plsc.addupdate_scatter(cd_loc, [di], ones)
        return 0

    lax.fori_loop(0, EPW // 16, body, 0)
    pltpu.sync_copy(cs_loc, out_hbm.at[0, w, 0])
    pltpu.sync_copy(cd_loc, out_hbm.at[1, w, 0])


@functools.cache
def _sc_counts():
    return pl.kernel(
        _sc_counts_body,
        out_type=jax.ShapeDtypeStruct((2, NW, 1, N), jnp.float32),
        mesh=_mesh(),
        compiler_params=pltpu.CompilerParams(needs_layout_passes=False),
        scratch_types=[
            pltpu.VMEM((EPW,), jnp.int32),
            pltpu.VMEM((EPW,), jnp.int32),
            pltpu.VMEM((N,), jnp.float32),
            pltpu.VMEM((N,), jnp.float32),
        ],
    )


def _sc_sf_body(src_hbm, dst_hbm, dinv_hbm, y_hbm, s_out, acc_out, acc_sh,
                dinv_loc, s_loc, srcrow, dstrow, yrow0, yrow1, zbuf,
                sem0, sem1):
    c = lax.axis_index("c")
    s = lax.axis_index("s")
    w = s * NC + c
    base = w * EPW
    pltpu.sync_copy(dinv_hbm, dinv_loc)

    zeros = jnp.zeros((16,), jnp.float32)

    def zs(i, _):
        s_loc[pl.ds(i * 16, 16)] = zeros
        return 0

    lax.fori_loop(0, N // 16, zs, 0)

    def zz(i, _):
        j = i // (D // 16)
        k = i % (D // 16)
        zbuf[j, pl.ds(k * 16, 16)] = zeros
        return 0

    lax.fori_loop(0, ZR * (D // 16), zz, 0)

    base_row = s * SPS
    for q in range(SPS // ZR):
        pltpu.sync_copy(zbuf, acc_sh.at[pl.ds(base_row + q * ZR, ZR)])
    plsc.subcore_barrier()

    def load_idx(cidx, slot):
        pltpu.sync_copy(src_hbm.at[pl.ds(base + cidx * CH, CH)],
                        srcrow.at[slot])
        pltpu.sync_copy(dst_hbm.at[pl.ds(base + cidx * CH, CH)],
                        dstrow.at[slot])

    def spass(slot):
        def sub(k, _):
            idxs = srcrow[slot, pl.ds(k * 16, 16)]
            vals = plsc.load_gather(dinv_loc, [idxs])
            idxd = dstrow[slot, pl.ds(k * 16, 16)]
            plsc.addupdate_scatter(s_loc, [idxd], vals)
            return 0

        lax.fori_loop(0, CH // 16, sub, 0)

    load_idx(0, 0)
    pltpu.async_copy(y_hbm.at[srcrow.at[0]], yrow0, sem0)

    def pair(m, _):
        c0 = 2 * m
        load_idx(c0 + 1, 1)
        pltpu.async_copy(y_hbm.at[srcrow.at[1]], yrow1, sem1)
        pltpu.make_async_copy(y_hbm.at[srcrow.at[0]], yrow0, sem0).wait()
        spass(0)
        pltpu.sync_copy(yrow0, acc_sh.at[dstrow.at[0]], add=True)
        load_idx(c0 + 2, 0)
        pltpu.async_copy(y_hbm.at[srcrow.at[0]], yrow0, sem0)
        pltpu.make_async_copy(y_hbm.at[srcrow.at[1]], yrow1, sem1).wait()
        spass(1)
        pltpu.sync_copy(yrow1, acc_sh.at[dstrow.at[1]], add=True)
        return 0

    lax.fori_loop(0, (NCHUNK - 1) // 2, pair, 0)
    pltpu.make_async_copy(y_hbm.at[srcrow.at[0]], yrow0, sem0).wait()
    spass(0)
    pltpu.sync_copy(yrow0, acc_sh.at[dstrow.at[0]], add=True)

    pltpu.sync_copy(s_loc, s_out.at[w, 0])
    plsc.subcore_barrier()
    pltpu.sync_copy(acc_sh.at[pl.ds(base_row, SPS)],
                    acc_out.at[c, pl.ds(base_row, SPS)])


@functools.cache
def _sc_sf():
    return pl.kernel(
        _sc_sf_body,
        out_type=(
            jax.ShapeDtypeStruct((NW, 1, N), jnp.float32),
            jax.ShapeDtypeStruct((NC, NP, D), jnp.float32),
        ),
        mesh=_mesh(),
        compiler_params=pltpu.CompilerParams(needs_layout_passes=False),
        scratch_types=[
            pltpu.VMEM_SHARED((NP, D), jnp.float32),
            pltpu.VMEM((N,), jnp.float32),
            pltpu.VMEM((N,), jnp.float32),
            pltpu.VMEM((2, CH), jnp.int32),
            pltpu.VMEM((2, CH), jnp.int32),
            pltpu.VMEM((CH, D), jnp.float32),
            pltpu.VMEM((CH, D), jnp.float32),
            pltpu.VMEM((ZR, D), jnp.float32),
            pltpu.SemaphoreType.DMA,
            pltpu.SemaphoreType.DMA,
        ],
    )


def _tc_b1_body(counts_ref, dinv_ref, dinv2_ref, invcnt_ref):
    counts = counts_ref[...]
    cs = jnp.sum(counts[0], axis=0)
    cd = jnp.sum(counts[1], axis=0)
    deg_g = cs + cd
    deg = jnp.sqrt(deg_g + 1e-9)
    dinv = 1.0 / (deg + 1e-9)
    dinv2 = lax.rsqrt(jnp.maximum(cd + 1.0, 1e-12))
    invcnt = 1.0 / jnp.maximum(cd, 1.0)
    dinv_ref[...] = jnp.transpose(dinv)
    dinv2_ref[...] = jnp.transpose(dinv2)
    invcnt_ref[...] = jnp.transpose(invcnt)


def _tc_b1(counts):
    return pl.pallas_call(
        _tc_b1_body,
        out_shape=(
            jax.ShapeDtypeStruct((N, 1), jnp.float32),
            jax.ShapeDtypeStruct((N, 1), jnp.float32),
            jax.ShapeDtypeStruct((N, 1), jnp.float32),
        ),
    )(counts)


_RB = 1000


def _tc_b2_body(x_ref, wc_ref, w0_ref, dinv2_ref, y_ref, xs_ref):
    xb = x_ref[...]
    xw = jnp.dot(xb, wc_ref[...], preferred_element_type=jnp.float32)
    y_ref[...] = xw * dinv2_ref[...]
    xs_ref[...] = jnp.dot(xb, w0_ref[...], preferred_element_type=jnp.float32)


def _tc_b2(x, W_conv, W0, dinv2_c):
    grid = (N // _RB,)
    return pl.pallas_call(
        _tc_b2_body,
        grid=grid,
        in_specs=[
            pl.BlockSpec((_RB, D), lambda i: (i, 0)),
            pl.BlockSpec((D, D), lambda i: (0, 0)),
            pl.BlockSpec((D, D), lambda i: (0, 0)),
            pl.BlockSpec((_RB, 1), lambda i: (i, 0)),
        ],
        out_specs=[
            pl.BlockSpec((_RB, D), lambda i: (i, 0)),
            pl.BlockSpec((_RB, D), lambda i: (i, 0)),
        ],
        out_shape=(
            jax.ShapeDtypeStruct((N, D), jnp.float32),
            jax.ShapeDtypeStruct((N, D), jnp.float32),
        ),
    )(x, W_conv, W0, dinv2_c)


def _tc_sred_body(s_ref, dinv_ref, invcnt_ref, nn_ref):
    s_col = jnp.transpose(jnp.sum(s_ref[...], axis=0))
    nn_ref[...] = dinv_ref[...] * s_col * invcnt_ref[...]


def _tc_sred(s_parts, dinv_c, invcnt_c):
    return pl.pallas_call(
        _tc_sred_body,
        out_shape=jax.ShapeDtypeStruct((N, 1), jnp.float32),
    )(s_parts, dinv_c, invcnt_c)


def _tc_d1_body(acc0_ref, acc1_ref, y_ref, xs_ref, x_ref,
                nn_ref, dinv2_ref, wt_ref, wih_ref,
                bconv_ref, bsum_ref, gir_ref, giz_ref, gin_ref):
    accs = acc0_ref[...] + acc1_ref[...]
    yb = y_ref[...]
    conv = dinv2_ref[...] * (accs + yb) + bconv_ref[...]
    h_neigh = jnp.maximum(conv * nn_ref[...], 0.0)
    xb = x_ref[...]
    hchg = jnp.dot(h_neigh - xb, wt_ref[...],
                   preferred_element_type=jnp.float32)
    hc = jax.nn.sigmoid(h_neigh + xs_ref[...] + hchg)
    gi = jnp.dot(hc, wih_ref[...],
                 preferred_element_type=jnp.float32) + bsum_ref[...]
    gir_ref[...] = 0.5 * gi[:, 0:GH]
    giz_ref[...] = 0.5 * gi[:, GH:2 * GH]
    gin_ref[...] = gi[:, 2 * GH:3 * GH]


def _tc_d1(acc0, acc1, y, xs, x, nn_c, dinv2_c, Wt, WihT, bconv2, bsum):
    grid = (N // _RB,)
    blk = pl.BlockSpec((_RB, D), lambda i: (i, 0))
    col = pl.BlockSpec((_RB, 1), lambda i: (i, 0))
    gout = pl.BlockSpec((_RB, GH), lambda i: (i, 0))
    return pl.pallas_call(
        _tc_d1_body,
        grid=grid,
        in_specs=[
            blk, blk, blk, blk, blk,
            col, col,
            pl.BlockSpec((D, D), lambda i: (0, 0)),
            pl.BlockSpec((D, 3 * GH), lambda i: (0, 0)),
            pl.BlockSpec((1, D), lambda i: (0, 0)),
            pl.BlockSpec((1, 3 * GH), lambda i: (0, 0)),
        ],
        out_specs=[gout, gout, gout],
        out_shape=(
            jax.ShapeDtypeStruct((N, GH), jnp.float32),
            jax.ShapeDtypeStruct((N, GH), jnp.float32),
            jax.ShapeDtypeStruct((N, GH), jnp.float32),
        ),
    )(acc0, acc1, y, xs, x, nn_c, dinv2_c, Wt, WihT, bconv2, bsum)


def _tc_d2_body(gir_ref, giz_ref, gin_ref, wr_ref, wz_ref, wn_ref, bhn_ref,
                out_ref):
    wr = wr_ref[...]
    wz = wz_ref[...]
    wn = wn_ref[...]
    bhn = bhn_ref[...]

    def step(t, h):
        ghr = jnp.dot(h, wr, preferred_element_type=jnp.float32)
        ghz = jnp.dot(h, wz, preferred_element_type=jnp.float32)
        q = jnp.dot(h, wn, preferred_element_type=jnp.float32) + bhn
        pre = gin_ref[pl.ds(t, 1), :] + q
        tr = jnp.tanh(gir_ref[pl.ds(t, 1), :] + ghr)
        tz = jnp.tanh(giz_ref[pl.ds(t, 1), :] + ghz)
        ng = jnp.tanh(pre + q * tr)
        z = 0.5 + 0.5 * tz
        h_new = ng + z * (h - ng)
        out_ref[pl.ds(t, 1), :] = h_new
        return h_new

    lax.fori_loop(0, N, step, jnp.zeros((1, GH), jnp.float32), unroll=4)


def _tc_d2(gir, giz, gin, Wr, Wz, Wn, bhn):
    return pl.pallas_call(
        _tc_d2_body,
        out_shape=jax.ShapeDtypeStruct((N, GH), jnp.float32),
    )(gir, giz, gin, Wr, Wz, Wn, bhn)


def kernel(x, edge_index, W_conv, b_conv, W0, Wt, W_ih, W_hh, b_ih, b_hh):
    src, dst = edge_index[0], edge_index[1]
    counts = _sc_counts()(src, dst)
    dinv_c, dinv2_c, invcnt_c = _tc_b1(counts)
    y, xs = _tc_b2(x, W_conv, W0, dinv2_c)
    s_parts, acc_parts = _sc_sf()(src, dst, dinv_c.reshape(N), y)

    WihT = W_ih.T
    bsum = jnp.concatenate(
        [b_ih[:2 * GH] + b_hh[:2 * GH], b_ih[2 * GH:]]).reshape(1, 3 * GH)
    bhn = 0.5 * b_hh[2 * GH:].reshape(1, GH)
    WhhT = W_hh.T
    Wr = 0.5 * WhhT[:, 0:GH]
    Wz = 0.5 * WhhT[:, GH:2 * GH]
    Wn = 0.5 * WhhT[:, 2 * GH:3 * GH]
    bconv2 = b_conv.reshape(1, D)

    nn_c = _tc_sred(s_parts, dinv_c, invcnt_c)
    gir, giz, gin = _tc_d1(acc_parts[0], acc_parts[1], y, xs, x, nn_c,
                           dinv2_c, Wt, WihT, bconv2, bsum)
    out = _tc_d2(gir, giz, gin, Wr, Wz, Wn, bhn)
    return out


# speedup vs baseline: 21.7976x; 1.0040x over previous
import functools
import jax
import jax.numpy as jnp
from jax import lax
from jax.experimental import pallas as pl
from jax.experimental.pallas import tpu as pltpu
from jax.experimental.pallas import tpu_sc as plsc

N = 10000
E = 320000
D = 128
GH = 32
NC = 2
NS = 16
NW = NC * NS
EPW = E // NW
CH = 80
NCHUNK = EPW // CH
NP = 10240
SPS = NP // NS
ZR = 32

@functools.cache
def _mesh():
    return plsc.VectorSubcoreMesh(
        core_axis_name="c", subcore_axis_name="s",
        num_cores=NC, num_subcores=NS)


def _sc_counts_body(src_hbm, dst_hbm, out_hbm, src_buf, dst_buf, cs_loc, cd_loc):
    c = lax.axis_index("c")
    s = lax.axis_index("s")
    w = s * NC + c
    base = w * EPW
    pltpu.sync_copy(src_hbm.at[pl.ds(base, EPW)], src_buf)
    pltpu.sync_copy(dst_hbm.at[pl.ds(base, EPW)], dst_buf)
    zeros = jnp.zeros((16,), jnp.float32)

    def zero_body(i, _):
        cs_loc[pl.ds(i * 16, 16)] = zeros
        cd_loc[pl.ds(i * 16, 16)] = zeros
        return 0

    lax.fori_loop(0, N // 16, zero_body, 0)
    ones = jnp.ones((16,), jnp.float32)

    def body(i, _):
        si = src_buf[pl.ds(i * 16, 16)]
        plsc.addupdate_scatter(cs_loc, [si], ones)
        di = dst_buf[pl.ds(i * 16, 16)]
        plsc.addupdate_scatter(cd_loc, [di], ones)
        return 0

    lax.fori_loop(0, EPW // 16, body, 0)
    pltpu.sync_copy(cs_loc, out_hbm.at[0, w, 0])
    pltpu.sync_copy(cd_loc, out_hbm.at[1, w, 0])


@functools.cache
def _sc_counts():
    return pl.kernel(
        _sc_counts_body,
        out_type=jax.ShapeDtypeStruct((2, NW, 1, N), jnp.float32),
        mesh=_mesh(),
        compiler_params=pltpu.CompilerParams(needs_layout_passes=False),
        scratch_types=[
            pltpu.VMEM((EPW,), jnp.int32),
            pltpu.VMEM((EPW,), jnp.int32),
            pltpu.VMEM((N,), jnp.float32),
            pltpu.VMEM((N,), jnp.float32),
        ],
    )


def _sc_sf_body(src_hbm, dst_hbm, dinv_hbm, y_hbm, s_out, acc_out, acc_sh,
                dinv_loc, s_loc, srcrow, dstrow, yrow0, yrow1, zbuf,
                sem0, sem1):
    c = lax.axis_index("c")
    s = lax.axis_index("s")
    w = s * NC + c
    base = w * EPW
    pltpu.sync_copy(dinv_hbm, dinv_loc)

    zeros = jnp.zeros((16,), jnp.float32)

    def zs(i, _):
        s_loc[pl.ds(i * 16, 16)] = zeros
        return 0

    lax.fori_loop(0, N // 16, zs, 0)

    def zz(i, _):
        j = i // (D // 16)
        k = i % (D // 16)
        zbuf[j, pl.ds(k * 16, 16)] = zeros
        return 0

    lax.fori_loop(0, ZR * (D // 16), zz, 0)

    base_row = s * SPS
    for q in range(SPS // ZR):
        pltpu.sync_copy(zbuf, acc_sh.at[pl.ds(base_row + q * ZR, ZR)])
    plsc.subcore_barrier()

    def load_idx(cidx, slot):
        pltpu.sync_copy(src_hbm.at[pl.ds(base + cidx * CH, CH)],
                        srcrow.at[slot])
        pltpu.sync_copy(dst_hbm.at[pl.ds(base + cidx * CH, CH)],
                        dstrow.at[slot])

    def spass(slot):
        def sub(k, _):
            idxs = srcrow[slot, pl.ds(k * 16, 16)]
            vals = plsc.load_gather(dinv_loc, [idxs])
            idxd = dstrow[slot, pl.ds(k * 16, 16)]
            plsc.addupdate_scatter(s_loc, [idxd], vals)
            return 0

        lax.fori_loop(0, CH // 16, sub, 0)

    load_idx(0, 0)
    pltpu.async_copy(y_hbm.at[srcrow.at[0]], yrow0, sem0)

    def pair(m, _):
        c0 = 2 * m
        load_idx(c0 + 1, 1)
        pltpu.async_copy(y_hbm.at[srcrow.at[1]], yrow1, sem1)
        pltpu.make_async_copy(y_hbm.at[srcrow.at[0]], yrow0, sem0).wait()
        spass(0)
        pltpu.sync_copy(yrow0, acc_sh.at[dstrow.at[0]], add=True)
        load_idx(c0 + 2, 0)
        pltpu.async_copy(y_hbm.at[srcrow.at[0]], yrow0, sem0)
        pltpu.make_async_copy(y_hbm.at[srcrow.at[1]], yrow1, sem1).wait()
        spass(1)
        pltpu.sync_copy(yrow1, acc_sh.at[dstrow.at[1]], add=True)
        return 0

    lax.fori_loop(0, (NCHUNK - 1) // 2, pair, 0)
    pltpu.make_async_copy(y_hbm.at[srcrow.at[0]], yrow0, sem0).wait()
    spass(0)
    pltpu.sync_copy(yrow0, acc_sh.at[dstrow.at[0]], add=True)

    pltpu.sync_copy(s_loc, s_out.at[w, 0])
    plsc.subcore_barrier()
    pltpu.sync_copy(acc_sh.at[pl.ds(base_row, SPS)],
                    acc_out.at[c, pl.ds(base_row, SPS)])


@functools.cache
def _sc_sf():
    return pl.kernel(
        _sc_sf_body,
        out_type=(
            jax.ShapeDtypeStruct((NW, 1, N), jnp.float32),
            jax.ShapeDtypeStruct((NC, NP, D), jnp.float32),
        ),
        mesh=_mesh(),
        compiler_params=pltpu.CompilerParams(needs_layout_passes=False),
        scratch_types=[
            pltpu.VMEM_SHARED((NP, D), jnp.float32),
            pltpu.VMEM((N,), jnp.float32),
            pltpu.VMEM((N,), jnp.float32),
            pltpu.VMEM((2, CH), jnp.int32),
            pltpu.VMEM((2, CH), jnp.int32),
            pltpu.VMEM((CH, D), jnp.float32),
            pltpu.VMEM((CH, D), jnp.float32),
            pltpu.VMEM((ZR, D), jnp.float32),
            pltpu.SemaphoreType.DMA,
            pltpu.SemaphoreType.DMA,
        ],
    )


def _tc_b1_body(counts_ref, dinv_ref, dinv2_ref, invcnt_ref):
    counts = counts_ref[...]
    cs = jnp.sum(counts[0], axis=0)
    cd = jnp.sum(counts[1], axis=0)
    deg_g = cs + cd
    deg = jnp.sqrt(deg_g + 1e-9)
    dinv = 1.0 / (deg + 1e-9)
    dinv2 = lax.rsqrt(jnp.maximum(cd + 1.0, 1e-12))
    invcnt = 1.0 / jnp.maximum(cd, 1.0)
    dinv_ref[...] = jnp.transpose(dinv)
    dinv2_ref[...] = jnp.transpose(dinv2)
    invcnt_ref[...] = jnp.transpose(invcnt)


def _tc_b1(counts):
    return pl.pallas_call(
        _tc_b1_body,
        out_shape=(
            jax.ShapeDtypeStruct((N, 1), jnp.float32),
            jax.ShapeDtypeStruct((N, 1), jnp.float32),
            jax.ShapeDtypeStruct((N, 1), jnp.float32),
        ),
    )(counts)


_RB = 1000


def _tc_b2_body(x_ref, wc_ref, w0_ref, dinv2_ref, y_ref, xs_ref):
    xb = x_ref[...]
    xw = jnp.dot(xb, wc_ref[...], preferred_element_type=jnp.float32)
    y_ref[...] = xw * dinv2_ref[...]
    xs_ref[...] = jnp.dot(xb, w0_ref[...], preferred_element_type=jnp.float32)


def _tc_b2(x, W_conv, W0, dinv2_c):
    grid = (N // _RB,)
    return pl.pallas_call(
        _tc_b2_body,
        grid=grid,
        in_specs=[
            pl.BlockSpec((_RB, D), lambda i: (i, 0)),
            pl.BlockSpec((D, D), lambda i: (0, 0)),
            pl.BlockSpec((D, D), lambda i: (0, 0)),
            pl.BlockSpec((_RB, 1), lambda i: (i, 0)),
        ],
        out_specs=[
            pl.BlockSpec((_RB, D), lambda i: (i, 0)),
            pl.BlockSpec((_RB, D), lambda i: (i, 0)),
        ],
        out_shape=(
            jax.ShapeDtypeStruct((N, D), jnp.float32),
            jax.ShapeDtypeStruct((N, D), jnp.float32),
        ),
    )(x, W_conv, W0, dinv2_c)


def _tc_sred_body(s_ref, dinv_ref, invcnt_ref, nn_ref):
    s_col = jnp.transpose(jnp.sum(s_ref[...], axis=0))
    nn_ref[...] = dinv_ref[...] * s_col * invcnt_ref[...]


def _tc_sred(s_parts, dinv_c, invcnt_c):
    return pl.pallas_call(
        _tc_sred_body,
        out_shape=jax.ShapeDtypeStruct((N, 1), jnp.float32),
    )(s_parts, dinv_c, invcnt_c)


def _tc_d1_body(acc0_ref, acc1_ref, y_ref, xs_ref, x_ref,
                nn_ref, dinv2_ref, wt_ref, wih_ref,
                bconv_ref, bsum_ref, gir_ref, giz_ref, gin_ref):
    accs = acc0_ref[...] + acc1_ref[...]
    yb = y_ref[...]
    conv = dinv2_ref[...] * (accs + yb) + bconv_ref[...]
    h_neigh = jnp.maximum(conv * nn_ref[...], 0.0)
    xb = x_ref[...]
    hchg = jnp.dot(h_neigh - xb, wt_ref[...],
                   preferred_element_type=jnp.float32)
    hc = jax.nn.sigmoid(h_neigh + xs_ref[...] + hchg)
    gi = jnp.dot(hc, wih_ref[...],
                 preferred_element_type=jnp.float32) + bsum_ref[...]
    gir_ref[...] = 0.5 * gi[:, 0:GH]
    giz_ref[...] = 0.5 * gi[:, GH:2 * GH]
    gin_ref[...] = gi[:, 2 * GH:3 * GH]


def _tc_d1(acc0, acc1, y, xs, x, nn_c, dinv2_c, Wt, WihT, bconv2, bsum):
    grid = (N // _RB,)
    blk = pl.BlockSpec((_RB, D), lambda i: (i, 0))
    col = pl.BlockSpec((_RB, 1), lambda i: (i, 0))
    gout = pl.BlockSpec((_RB, GH), lambda i: (i, 0))
    return pl.pallas_call(
        _tc_d1_body,
        grid=grid,
        in_specs=[
            blk, blk, blk, blk, blk,
            col, col,
            pl.BlockSpec((D, D), lambda i: (0, 0)),
            pl.BlockSpec((D, 3 * GH), lambda i: (0, 0)),
            pl.BlockSpec((1, D), lambda i: (0, 0)),
            pl.BlockSpec((1, 3 * GH), lambda i: (0, 0)),
        ],
        out_specs=[gout, gout, gout],
        out_shape=(
            jax.ShapeDtypeStruct((N, GH), jnp.float32),
            jax.ShapeDtypeStruct((N, GH), jnp.float32),
            jax.ShapeDtypeStruct((N, GH), jnp.float32),
        ),
    )(acc0, acc1, y, xs, x, nn_c, dinv2_c, Wt, WihT, bconv2, bsum)


def _tc_d2_body(gir_ref, giz_ref, gin_ref, wr_ref, wz_ref, wn_ref, bhn_ref,
                out_ref):
    wr = wr_ref[...]
    wz = wz_ref[...]
    wn = wn_ref[...]
    bhn = bhn_ref[...]

    def step(t, h):
        ghr = jnp.dot(h, wr, preferred_element_type=jnp.float32)
        ghz = jnp.dot(h, wz, preferred_element_type=jnp.float32)
        q = jnp.dot(h, wn, preferred_element_type=jnp.float32) + bhn
        pre = gin_ref[pl.ds(t, 1), :] + q
        tr = jnp.tanh(gir_ref[pl.ds(t, 1), :] + ghr)
        tz = jnp.tanh(giz_ref[pl.ds(t, 1), :] + ghz)
        ng = jnp.tanh(pre + q * tr)
        z = 0.5 + 0.5 * tz
        h_new = ng + z * (h - ng)
        out_ref[pl.ds(t, 1), :] = h_new
        return h_new

    lax.fori_loop(0, N, step, jnp.zeros((1, GH), jnp.float32), unroll=8)


def _tc_d2(gir, giz, gin, Wr, Wz, Wn, bhn):
    return pl.pallas_call(
        _tc_d2_body,
        out_shape=jax.ShapeDtypeStruct((N, GH), jnp.float32),
    )(gir, giz, gin, Wr, Wz, Wn, bhn)


def kernel(x, edge_index, W_conv, b_conv, W0, Wt, W_ih, W_hh, b_ih, b_hh):
    src, dst = edge_index[0], edge_index[1]
    counts = _sc_counts()(src, dst)
    dinv_c, dinv2_c, invcnt_c = _tc_b1(counts)
    y, xs = _tc_b2(x, W_conv, W0, dinv2_c)
    s_parts, acc_parts = _sc_sf()(src, dst, dinv_c.reshape(N), y)

    WihT = W_ih.T
    bsum = jnp.concatenate(
        [b_ih[:2 * GH] + b_hh[:2 * GH], b_ih[2 * GH:]]).reshape(1, 3 * GH)
    bhn = 0.5 * b_hh[2 * GH:].reshape(1, GH)
    WhhT = W_hh.T
    Wr = 0.5 * WhhT[:, 0:GH]
    Wz = 0.5 * WhhT[:, GH:2 * GH]
    Wn = 0.5 * WhhT[:, 2 * GH:3 * GH]
    bconv2 = b_conv.reshape(1, D)

    nn_c = _tc_sred(s_parts, dinv_c, invcnt_c)
    gir, giz, gin = _tc_d1(acc_parts[0], acc_parts[1], y, xs, x, nn_c,
                           dinv2_c, Wt, WihT, bconv2, bsum)
    out = _tc_d2(gir, giz, gin, Wr, Wz, Wn, bhn)
    return out
